# baseline (speedup 1.0000x reference)
import functools
import math

import jax
import jax.numpy as jnp
from jax import lax
from jax.experimental import pallas as pl
from jax.experimental.pallas import tpu as pltpu
from jax.experimental.pallas import tpu_sc as plsc

VOCAB = 1000000
MAX_POS = 512
HIDDEN = 64
BATCH = 4096
SEQ = 50

NC = 2
NS = 16
NW = NC * NS
LANES = 16

SEQS_PER_W = BATCH // NW
SEQS_PER_CHUNK = 16
NCHUNK = SEQS_PER_W // SEQS_PER_CHUNK
VPR = HIDDEN // LANES
SEQ_PAD = 56
RBLOCK = 25


def _emb_kernel(ids_hbm, table_hbm, pos_hbm, out_hbm,
                idx0, idx1, rows0, rows1, pos_v,
                gsem0, gsem1, wsem0, wsem1):
    wid = lax.axis_index("s") * NC + lax.axis_index("c")
    wseq = wid * SEQS_PER_W

    idx_v = [idx0, idx1]
    rows_v = [rows0, rows1]
    gsem = [gsem0, gsem1]
    wsem = [wsem0, wsem1]

    pltpu.sync_copy(pos_hbm.at[pl.ds(0, SEQ_PAD)], pos_v)
    scale = jnp.float32(1.0 / math.sqrt(HIDDEN))

    def _scale_row(r, _):
        for q in range(VPR):
            pos_v[r, pl.ds(q * LANES, LANES)] = (
                pos_v[r, pl.ds(q * LANES, LANES)] * scale)
        return 0

    lax.fori_loop(0, SEQ, _scale_row, 0)

    def _add_pos(buf):
        def _block(b, _):
            s = b // 2
            r0 = (b % 2) * RBLOCK
            for r in range(RBLOCK):
                for q in range(VPR):
                    sl = pl.ds(q * LANES, LANES)
                    buf[s, r0 + r, sl] = buf[s, r0 + r, sl] + pos_v[r0 + r, sl]
            return 0

        lax.fori_loop(0, SEQS_PER_CHUNK * 2, _block, 0)

    def _start_gathers(c):
        b = c % 2
        s0 = wseq + c * SEQS_PER_CHUNK
        pltpu.sync_copy(ids_hbm.at[pl.ds(s0, SEQS_PER_CHUNK)], idx_v[b])
        copies = []
        for s in range(SEQS_PER_CHUNK):
            copies.append(pltpu.async_copy(
                table_hbm.at[idx_v[b].at[s]], rows_v[b].at[s], gsem[b]))
        return copies

    writes = [None, None]
    gathers = [None, None]
    gathers[0] = _start_gathers(0)
    for c in range(NCHUNK):
        b = c % 2
        nb = (c + 1) % 2
        if c + 1 < NCHUNK:
            if writes[nb] is not None:
                writes[nb].wait()
                writes[nb] = None
            gathers[nb] = _start_gathers(c + 1)
        for g in gathers[b]:
            g.wait()
        _add_pos(rows_v[b])
        s0 = wseq + c * SEQS_PER_CHUNK
        writes[b] = pltpu.async_copy(
            rows_v[b], out_hbm.at[pl.ds(s0, SEQS_PER_CHUNK)], wsem[b])
    for w in writes:
        if w is not None:
            w.wait()


@jax.jit
def _emb(input_ids, ids_table, pos_table):
    mesh = plsc.VectorSubcoreMesh(core_axis_name="c", subcore_axis_name="s")
    f = pl.kernel(
        _emb_kernel,
        out_type=jax.ShapeDtypeStruct((BATCH, SEQ, HIDDEN), jnp.float32),
        mesh=mesh,
        scratch_types=[
            pltpu.VMEM((SEQS_PER_CHUNK, SEQ), jnp.int32),
            pltpu.VMEM((SEQS_PER_CHUNK, SEQ), jnp.int32),
            pltpu.VMEM((SEQS_PER_CHUNK, SEQ, HIDDEN), jnp.float32),
            pltpu.VMEM((SEQS_PER_CHUNK, SEQ, HIDDEN), jnp.float32),
            pltpu.VMEM((SEQ_PAD, HIDDEN), jnp.float32),
            pltpu.SemaphoreType.DMA,
            pltpu.SemaphoreType.DMA,
            pltpu.SemaphoreType.DMA,
            pltpu.SemaphoreType.DMA,
        ],
        compiler_params=pltpu.CompilerParams(use_tc_tiling_on_sc=False),
    )
    return f(input_ids, ids_table, pos_table)


def kernel(input_ids, ids_table, pos_table):
    return _emb(input_ids, ids_table, pos_table)
